# baseline (speedup 1.0000x reference)
import jax
import jax.numpy as jnp
from jax import lax
from jax.experimental import pallas as pl
from jax.experimental.pallas import tpu as pltpu
from jax.experimental.pallas import tpu_sc as plsc

N = 10000
E = 320000
D = 128

NC = 2
NS = 16
L = 16
NW = NC * NS
EW = E // NW
CB = 80
NCHUNK = EW // CB
NB = 4
GA = 3
NBI = 8
IA = 5
UNROLL = 8
NOUTER = (NCHUNK + UNROLL - 1) // UNROLL
RT = 624
TAIL = N - RT * NS
DRAIN = sorted({i % NB for i in range(NCHUNK - (NB - GA), NCHUNK)})


def _spmm_body(vecs, row3, col3, val3, out,
               rowr, colr, valr, rows, acc, semi, semg, sems):
    cid = lax.axis_index("c")
    sid = lax.axis_index("s")
    wid = cid * NS + sid

    def edata_dmas(c, d):
        return (pltpu.make_async_copy(row3.at[wid, c], rowr.at[d], semi.at[d]),
                pltpu.make_async_copy(col3.at[wid, c], colr.at[d], semi.at[d]),
                pltpu.make_async_copy(val3.at[wid, c], valr.at[d], semi.at[d]))

    def gather(b, d):
        return pltpu.make_async_copy(vecs.at[colr.at[d]], rows.at[b],
                                     semg.at[b])

    def scatter(b, d):
        return pltpu.make_async_copy(rows.at[b], acc.at[rowr.at[d]],
                                     sems.at[b])

    for d in range(IA):
        for dma in edata_dmas(d, d):
            dma.start()

    def zrow(i, _):
        for j in range(D // L):
            rows[0, i, pl.ds(j * L, L)] = jnp.zeros((L,), jnp.float32)
        return 0

    lax.fori_loop(0, CB, zrow, 0)
    for k in range(RT // CB):
        pltpu.sync_copy(rows.at[0], acc.at[pl.ds(sid * RT + k * CB, CB)])
    rem = RT % CB
    pltpu.sync_copy(rows.at[0, pl.ds(0, rem)],
                    acc.at[pl.ds(sid * RT + (RT // CB) * CB, rem)])

    @pl.when(sid == NS - 1)
    def _():
        pltpu.sync_copy(rows.at[0, pl.ds(0, TAIL)],
                        acc.at[pl.ds(RT * NS, TAIL)])

    plsc.subcore_barrier()

    for c in range(GA):
        for dma in edata_dmas(c, c):
            dma.wait()
        gather(c, c).start()

    def outer(g, _):
        for u in range(UNROLL):
            i = g * UNROLL + u
            b = u % NB
            d = u % NBI
            b2 = (u + GA) % NB
            d2 = (u + GA) % NBI

            @pl.when(i + GA < NCHUNK)
            def _():
                for dma in edata_dmas(i + GA, d2):
                    dma.wait()

                @pl.when(i + GA >= NB)
                def _():
                    scatter(b2, d2).wait()

                gather(b2, d2).start()

            @pl.when(i + IA < NCHUNK)
            def _():
                for dma in edata_dmas(i + IA, (u + IA) % NBI):
                    dma.start()

            @pl.when(i < NCHUNK)
            def _():
                gather(b, d).wait()

                def scale(gi, _):
                    val16 = valr[d, pl.ds(gi * L, L)]
                    for e in range(L):
                        r = gi * L + e
                        vb = jnp.full((L,), val16[e], jnp.float32)
                        for j in range(D // L):
                            sl = pl.ds(j * L, L)
                            rows[b, r, sl] = rows[b, r, sl] * vb
                    return 0

                lax.fori_loop(0, CB // L, scale, 0)

                pltpu.async_copy(rows.at[b], acc.at[rowr.at[d]],
                                 sems.at[b], add=True)
        return 0

    lax.fori_loop(0, NOUTER, outer, 0)

    for b in DRAIN:
        scatter(b, 0).wait()
    plsc.subcore_barrier()

    pltpu.sync_copy(acc.at[pl.ds(sid * RT, RT)], out.at[cid, pl.ds(sid * RT, RT)])

    @pl.when(sid == NS - 1)
    def _():
        pltpu.sync_copy(acc.at[pl.ds(RT * NS, TAIL)],
                        out.at[cid, pl.ds(RT * NS, TAIL)])


_spmm = pl.kernel(
    _spmm_body,
    out_type=jax.ShapeDtypeStruct((NC, N, D), jnp.float32),
    mesh=plsc.VectorSubcoreMesh(
        core_axis_name="c", subcore_axis_name="s", num_cores=NC, num_subcores=NS
    ),
    scratch_types=[
        pltpu.VMEM((NBI, CB), jnp.int32),
        pltpu.VMEM((NBI, CB), jnp.int32),
        pltpu.VMEM((NBI, CB), jnp.float32),
        pltpu.VMEM((NB, CB, D), jnp.float32),
        pltpu.VMEM_SHARED((N, D), jnp.float32),
        pltpu.SemaphoreType.DMA((NBI,)),
        pltpu.SemaphoreType.DMA((NB,)),
        pltpu.SemaphoreType.DMA((NB,)),
    ],
)


def _dense_body(x_ref, p_ref, w0_ref, b0_ref, off0_ref, sc0_ref,
                w1_ref, b1_ref, off1_ref, sc1_ref, o_ref):
    def branch(v, w, b, off, sc):
        vw = jnp.dot(v, w, preferred_element_type=jnp.float32) + b
        vw = jnp.maximum(vw, 0.0)
        mean = jnp.mean(vw, axis=1, keepdims=True)
        var = jnp.mean(jnp.square(vw - mean), axis=1, keepdims=True)
        return sc * (vw - mean) * lax.rsqrt(var + 1e-9) + off

    h1 = p_ref[0] + p_ref[1]
    o_ref[...] = (
        branch(x_ref[...], w0_ref[...], b0_ref[...], off0_ref[...], sc0_ref[...])
        + branch(h1, w1_ref[...], b1_ref[...], off1_ref[...], sc1_ref[...])
    )


BLK = 1000


def _dense(vecs, partial, W0, b0, off0, sc0, W1, b1, off1, sc1):
    full = lambda shape: pl.BlockSpec(shape, lambda i: (0,) * len(shape))
    return pl.pallas_call(
        _dense_body,
        grid=(N // BLK,),
        in_specs=[
            pl.BlockSpec((BLK, D), lambda i: (i, 0)),
            pl.BlockSpec((NC, BLK, D), lambda i: (0, i, 0)),
            full((D, D)), full((1, D)), full((1, D)), full((1, D)),
            full((D, D)), full((1, D)), full((1, D)), full((1, D)),
        ],
        out_specs=pl.BlockSpec((BLK, D), lambda i: (i, 0)),
        out_shape=jax.ShapeDtypeStruct((N, D), jnp.float32),
    )(vecs, partial, W0, b0, off0, sc0, W1, b1, off1, sc1)


def kernel(vecs, adj_indices, adj_values, W0, b0, off0, sc0, W1, b1, off1, sc1):
    row3 = adj_indices[0].reshape(NW, NCHUNK, CB)
    col3 = adj_indices[1].reshape(NW, NCHUNK, CB)
    val3 = adj_values.reshape(NW, NCHUNK, CB)
    partial = _spmm(vecs, row3, col3, val3)
    return _dense(vecs, partial, W0, b0.reshape(1, D), off0, sc0,
                  W1, b1.reshape(1, D), off1, sc1)


# speedup vs baseline: 1.0246x; 1.0246x over previous
import jax
import jax.numpy as jnp
from jax import lax
from jax.experimental import pallas as pl
from jax.experimental.pallas import tpu as pltpu
from jax.experimental.pallas import tpu_sc as plsc

N = 10000
E = 320000
D = 128

NC = 2
NS = 16
L = 16
NW = NC * NS
EW = E // NW
CB = 80
NCHUNK = EW // CB
NB = 4
GA = 2
NBI = 8
IA = 5
UNROLL = 8
NOUTER = (NCHUNK + UNROLL - 1) // UNROLL
RT = 624
TAIL = N - RT * NS
DRAIN = sorted({i % NB for i in range(NCHUNK - (NB - GA), NCHUNK)})


def _spmm_body(vecs, row3, col3, val3, out,
               rowr, colr, valr, rows, acc, semi, semg, sems):
    cid = lax.axis_index("c")
    sid = lax.axis_index("s")
    wid = cid * NS + sid

    def edata_dmas(c, d):
        return (pltpu.make_async_copy(row3.at[wid, c], rowr.at[d], semi.at[d]),
                pltpu.make_async_copy(col3.at[wid, c], colr.at[d], semi.at[d]),
                pltpu.make_async_copy(val3.at[wid, c], valr.at[d], semi.at[d]))

    def gather(b, d):
        return pltpu.make_async_copy(vecs.at[colr.at[d]], rows.at[b],
                                     semg.at[b])

    def scatter(b, d):
        return pltpu.make_async_copy(rows.at[b], acc.at[rowr.at[d]],
                                     sems.at[b])

    for d in range(IA):
        for dma in edata_dmas(d, d):
            dma.start()

    def zrow(i, _):
        for j in range(D // L):
            rows[0, i, pl.ds(j * L, L)] = jnp.zeros((L,), jnp.float32)
        return 0

    lax.fori_loop(0, CB, zrow, 0)
    for k in range(RT // CB):
        pltpu.sync_copy(rows.at[0], acc.at[pl.ds(sid * RT + k * CB, CB)])
    rem = RT % CB
    pltpu.sync_copy(rows.at[0, pl.ds(0, rem)],
                    acc.at[pl.ds(sid * RT + (RT // CB) * CB, rem)])

    @pl.when(sid == NS - 1)
    def _():
        pltpu.sync_copy(rows.at[0, pl.ds(0, TAIL)],
                        acc.at[pl.ds(RT * NS, TAIL)])

    plsc.subcore_barrier()

    for c in range(GA):
        for dma in edata_dmas(c, c):
            dma.wait()
        gather(c, c).start()

    def outer(g, _):
        for u in range(UNROLL):
            i = g * UNROLL + u
            b = u % NB
            d = u % NBI
            b2 = (u + GA) % NB
            d2 = (u + GA) % NBI

            @pl.when(i + GA < NCHUNK)
            def _():
                for dma in edata_dmas(i + GA, d2):
                    dma.wait()

                @pl.when(i + GA >= NB)
                def _():
                    scatter(b2, d2).wait()

                gather(b2, d2).start()

            @pl.when(i + IA < NCHUNK)
            def _():
                for dma in edata_dmas(i + IA, (u + IA) % NBI):
                    dma.start()

            @pl.when(i < NCHUNK)
            def _():
                gather(b, d).wait()

                def scale(gi, _):
                    val16 = valr[d, pl.ds(gi * L, L)]
                    for e in range(L):
                        r = gi * L + e
                        vb = jnp.full((L,), val16[e], jnp.float32)
                        for j in range(D // L):
                            sl = pl.ds(j * L, L)
                            rows[b, r, sl] = rows[b, r, sl] * vb
                    return 0

                lax.fori_loop(0, CB // L, scale, 0)

                pltpu.async_copy(rows.at[b], acc.at[rowr.at[d]],
                                 sems.at[b], add=True)
        return 0

    lax.fori_loop(0, NOUTER, outer, 0)

    for b in DRAIN:
        scatter(b, 0).wait()
    plsc.subcore_barrier()

    pltpu.sync_copy(acc.at[pl.ds(sid * RT, RT)], out.at[cid, pl.ds(sid * RT, RT)])

    @pl.when(sid == NS - 1)
    def _():
        pltpu.sync_copy(acc.at[pl.ds(RT * NS, TAIL)],
                        out.at[cid, pl.ds(RT * NS, TAIL)])


_spmm = pl.kernel(
    _spmm_body,
    out_type=jax.ShapeDtypeStruct((NC, N, D), jnp.float32),
    mesh=plsc.VectorSubcoreMesh(
        core_axis_name="c", subcore_axis_name="s", num_cores=NC, num_subcores=NS
    ),
    scratch_types=[
        pltpu.VMEM((NBI, CB), jnp.int32),
        pltpu.VMEM((NBI, CB), jnp.int32),
        pltpu.VMEM((NBI, CB), jnp.float32),
        pltpu.VMEM((NB, CB, D), jnp.float32),
        pltpu.VMEM_SHARED((N, D), jnp.float32),
        pltpu.SemaphoreType.DMA((NBI,)),
        pltpu.SemaphoreType.DMA((NB,)),
        pltpu.SemaphoreType.DMA((NB,)),
    ],
)


def _branch(v, w, b, off, sc):
    vw = jnp.dot(v, w, preferred_element_type=jnp.float32) + b
    vw = jnp.maximum(vw, 0.0)
    mean = jnp.mean(vw, axis=1, keepdims=True)
    var = jnp.mean(jnp.square(vw - mean), axis=1, keepdims=True)
    return sc * (vw - mean) * lax.rsqrt(var + 1e-9) + off


def _v0_body(x_ref, w0_ref, b0_ref, off0_ref, sc0_ref, o_ref):
    o_ref[...] = _branch(x_ref[...], w0_ref[...], b0_ref[...],
                         off0_ref[...], sc0_ref[...])


def _v1_body(v0_ref, p_ref, w1_ref, b1_ref, off1_ref, sc1_ref, o_ref):
    h1 = p_ref[0] + p_ref[1]
    o_ref[...] = v0_ref[...] + _branch(h1, w1_ref[...], b1_ref[...],
                                       off1_ref[...], sc1_ref[...])


BLK = 1000


def _dense_v0(vecs, W0, b0, off0, sc0):
    full = lambda shape: pl.BlockSpec(shape, lambda i: (0,) * len(shape))
    return pl.pallas_call(
        _v0_body,
        grid=(N // BLK,),
        in_specs=[
            pl.BlockSpec((BLK, D), lambda i: (i, 0)),
            full((D, D)), full((1, D)), full((1, D)), full((1, D)),
        ],
        out_specs=pl.BlockSpec((BLK, D), lambda i: (i, 0)),
        out_shape=jax.ShapeDtypeStruct((N, D), jnp.float32),
    )(vecs, W0, b0, off0, sc0)


def _dense_v1(v0, partial, W1, b1, off1, sc1):
    full = lambda shape: pl.BlockSpec(shape, lambda i: (0,) * len(shape))
    return pl.pallas_call(
        _v1_body,
        grid=(N // BLK,),
        in_specs=[
            pl.BlockSpec((BLK, D), lambda i: (i, 0)),
            pl.BlockSpec((NC, BLK, D), lambda i: (0, i, 0)),
            full((D, D)), full((1, D)), full((1, D)), full((1, D)),
        ],
        out_specs=pl.BlockSpec((BLK, D), lambda i: (i, 0)),
        out_shape=jax.ShapeDtypeStruct((N, D), jnp.float32),
    )(v0, partial, W1, b1, off1, sc1)


def kernel(vecs, adj_indices, adj_values, W0, b0, off0, sc0, W1, b1, off1, sc1):
    row3 = adj_indices[0].reshape(NW, NCHUNK, CB)
    col3 = adj_indices[1].reshape(NW, NCHUNK, CB)
    val3 = adj_values.reshape(NW, NCHUNK, CB)
    partial = _spmm(vecs, row3, col3, val3)
    v0 = _dense_v0(vecs, W0, b0.reshape(1, D), off0, sc0)
    return _dense_v1(v0, partial, W1, b1.reshape(1, D), off1, sc1)


# speedup vs baseline: 1.0373x; 1.0124x over previous
import jax
import jax.numpy as jnp
from jax import lax
from jax.experimental import pallas as pl
from jax.experimental.pallas import tpu as pltpu
from jax.experimental.pallas import tpu_sc as plsc

N = 10000
E = 320000
D = 128

NC = 2
NS = 16
L = 16
NW = NC * NS
EW = E // NW
CB = 80
NCHUNK = EW // CB
NB = 4
GA = 2
NBI = 8
IA = 5
UNROLL = 8
NOUTER = (NCHUNK + UNROLL - 1) // UNROLL
RT = 624
TAIL = N - RT * NS
DRAIN = list(range(NB))


def _spmm_body(vecs, row3, col3, val3, out,
               rowr, colr, valr, rows, acc, semi, semg, sems):
    cid = lax.axis_index("c")
    sid = lax.axis_index("s")
    wid = cid * NS + sid

    def edata_dmas(c, d):
        return (pltpu.make_async_copy(row3.at[wid, c], rowr.at[d], semi.at[d]),
                pltpu.make_async_copy(col3.at[wid, c], colr.at[d], semi.at[d]),
                pltpu.make_async_copy(val3.at[wid, c], valr.at[d], semi.at[d]))

    def gather(b, d):
        return pltpu.make_async_copy(vecs.at[colr.at[d]], rows.at[b],
                                     semg.at[b])

    def scatter(b, d):
        return pltpu.make_async_copy(rows.at[b], acc.at[rowr.at[d]],
                                     sems.at[b])

    for d in range(IA):
        for dma in edata_dmas(d, d):
            dma.start()

    def zrow(i, _):
        for j in range(D // L):
            rows[0, i, pl.ds(j * L, L)] = jnp.zeros((L,), jnp.float32)
        return 0

    lax.fori_loop(0, CB, zrow, 0)
    for k in range(RT // CB):
        pltpu.sync_copy(rows.at[0], acc.at[pl.ds(sid * RT + k * CB, CB)])
    rem = RT % CB
    pltpu.sync_copy(rows.at[0, pl.ds(0, rem)],
                    acc.at[pl.ds(sid * RT + (RT // CB) * CB, rem)])

    @pl.when(sid == NS - 1)
    def _():
        pltpu.sync_copy(rows.at[0, pl.ds(0, TAIL)],
                        acc.at[pl.ds(RT * NS, TAIL)])

    plsc.subcore_barrier()

    for c in range(GA):
        for dma in edata_dmas(c, c):
            dma.wait()
        gather(c, c).start()

    def outer(g, _):
        for u in range(UNROLL):
            i = g * UNROLL + u
            b = u % NB
            d = u % NBI
            b2 = (u + GA) % NB
            d2 = (u + GA) % NBI

            @pl.when(i + GA < NCHUNK)
            def _():
                for dma in edata_dmas(i + GA, d2):
                    dma.wait()

                @pl.when(i + GA >= NB)
                def _():
                    scatter(b2, d2).wait()

                gather(b2, d2).start()

            @pl.when(i + IA < NCHUNK)
            def _():
                for dma in edata_dmas(i + IA, (u + IA) % NBI):
                    dma.start()

            @pl.when(i < NCHUNK)
            def _():
                gather(b, d).wait()

                def scale(gi, _):
                    val16 = valr[d, pl.ds(gi * L, L)]
                    for e in range(L):
                        r = gi * L + e
                        vb = jnp.full((L,), val16[e], jnp.float32)
                        for j in range(D // L):
                            sl = pl.ds(j * L, L)
                            rows[b, r, sl] = rows[b, r, sl] * vb
                    return 0

                lax.fori_loop(0, CB // L, scale, 0)

                pltpu.async_copy(rows.at[b], acc.at[rowr.at[d]],
                                 sems.at[b], add=True)
        return 0

    lax.fori_loop(0, NOUTER, outer, 0)

    for b in DRAIN:
        scatter(b, 0).wait()
    plsc.subcore_barrier()

    pltpu.sync_copy(acc.at[pl.ds(sid * RT, RT)], out.at[cid, pl.ds(sid * RT, RT)])

    @pl.when(sid == NS - 1)
    def _():
        pltpu.sync_copy(acc.at[pl.ds(RT * NS, TAIL)],
                        out.at[cid, pl.ds(RT * NS, TAIL)])


_spmm = pl.kernel(
    _spmm_body,
    out_type=jax.ShapeDtypeStruct((NC, N, D), jnp.float32),
    mesh=plsc.VectorSubcoreMesh(
        core_axis_name="c", subcore_axis_name="s", num_cores=NC, num_subcores=NS
    ),
    scratch_types=[
        pltpu.VMEM((NBI, CB), jnp.int32),
        pltpu.VMEM((NBI, CB), jnp.int32),
        pltpu.VMEM((NBI, CB), jnp.float32),
        pltpu.VMEM((NB, CB, D), jnp.float32),
        pltpu.VMEM_SHARED((N, D), jnp.float32),
        pltpu.SemaphoreType.DMA((NBI,)),
        pltpu.SemaphoreType.DMA((NB,)),
        pltpu.SemaphoreType.DMA((NB,)),
    ],
)


def _branch(v, w, b, off, sc):
    vw = jnp.dot(v, w, preferred_element_type=jnp.float32) + b
    vw = jnp.maximum(vw, 0.0)
    mean = jnp.mean(vw, axis=1, keepdims=True)
    var = jnp.mean(jnp.square(vw - mean), axis=1, keepdims=True)
    return sc * (vw - mean) * lax.rsqrt(var + 1e-9) + off


def _v0_body(x_ref, w0_ref, b0_ref, off0_ref, sc0_ref, o_ref):
    o_ref[...] = _branch(x_ref[...], w0_ref[...], b0_ref[...],
                         off0_ref[...], sc0_ref[...])


def _v1_body(v0_ref, p_ref, w1_ref, b1_ref, off1_ref, sc1_ref, o_ref):
    h1 = p_ref[0] + p_ref[1]
    o_ref[...] = v0_ref[...] + _branch(h1, w1_ref[...], b1_ref[...],
                                       off1_ref[...], sc1_ref[...])


BLK = 1000


def _dense_v0(vecs, W0, b0, off0, sc0):
    full = lambda shape: pl.BlockSpec(shape, lambda i: (0,) * len(shape))
    return pl.pallas_call(
        _v0_body,
        grid=(N // BLK,),
        in_specs=[
            pl.BlockSpec((BLK, D), lambda i: (i, 0)),
            full((D, D)), full((1, D)), full((1, D)), full((1, D)),
        ],
        out_specs=pl.BlockSpec((BLK, D), lambda i: (i, 0)),
        out_shape=jax.ShapeDtypeStruct((N, D), jnp.float32),
    )(vecs, W0, b0, off0, sc0)


def _dense_v1(v0, partial, W1, b1, off1, sc1):
    full = lambda shape: pl.BlockSpec(shape, lambda i: (0,) * len(shape))
    return pl.pallas_call(
        _v1_body,
        grid=(N // BLK,),
        in_specs=[
            pl.BlockSpec((BLK, D), lambda i: (i, 0)),
            pl.BlockSpec((NC, BLK, D), lambda i: (0, i, 0)),
            full((D, D)), full((1, D)), full((1, D)), full((1, D)),
        ],
        out_specs=pl.BlockSpec((BLK, D), lambda i: (i, 0)),
        out_shape=jax.ShapeDtypeStruct((N, D), jnp.float32),
    )(v0, partial, W1, b1, off1, sc1)


def kernel(vecs, adj_indices, adj_values, W0, b0, off0, sc0, W1, b1, off1, sc1):
    row3 = adj_indices[0].reshape(NW, NCHUNK, CB)
    col3 = adj_indices[1].reshape(NW, NCHUNK, CB)
    val3 = adj_values.reshape(NW, NCHUNK, CB)
    partial = _spmm(vecs, row3, col3, val3)
    v0 = _dense_v0(vecs, W0, b0.reshape(1, D), off0, sc0)
    return _dense_v1(v0, partial, W1, b1.reshape(1, D), off1, sc1)
